# baseline (speedup 1.0000x reference)
import functools

import jax
import jax.numpy as jnp
from jax import lax
from jax.experimental import pallas as pl
from jax.experimental.pallas import tpu as pltpu
from jax.experimental.pallas import tpu_sc as plsc

B = 16384
EPAD = 32
CW = 128
NLANE = 16


def _sc_gather(x2d, hf, pf2, ep):
    info = plsc.get_sparse_core_info()
    nw = info.num_cores * info.num_subcores
    cpw = B // nw
    nch = cpw // CW
    nc = info.num_cores

    mesh = plsc.VectorSubcoreMesh(core_axis_name="c", subcore_axis_name="s")

    scratch = (
        [pltpu.VMEM((CW,), jnp.int32) for _ in range(nch)]
        + [pltpu.VMEM((CW,), jnp.int32) for _ in range(2 * nch)]
        + [pltpu.VMEM((CW,), jnp.int32) for _ in range(2 * nch)]
        + [pltpu.VMEM((CW, 1), jnp.float32) for _ in range(2 * nch)]
        + [pltpu.VMEM((cpw, EPAD), jnp.float32),
           pltpu.VMEM((cpw, EPAD), jnp.float32),
           pltpu.SemaphoreType.DMA]
    )

    @functools.partial(
        pl.kernel,
        out_type=(
            jax.ShapeDtypeStruct((B, EPAD), jnp.float32),
            jax.ShapeDtypeStruct((B, EPAD), jnp.float32),
            jax.ShapeDtypeStruct((B, 1), jnp.float32),
            jax.ShapeDtypeStruct((B, 1), jnp.float32),
        ),
        mesh=mesh,
        scratch_types=scratch,
        compiler_params=pltpu.CompilerParams(use_tc_tiling_on_sc=False),
    )
    def body(x_hbm, hf_hbm, pf2_hbm, ep_hbm,
             r0_out, r1_out, q0_out, q1_out, *scr):
        xv = scr[0:nch]
        i0 = scr[nch:2 * nch]
        i1 = scr[2 * nch:3 * nch]
        b0 = scr[3 * nch:4 * nch]
        b1 = scr[4 * nch:5 * nch]
        q0 = scr[5 * nch:6 * nch]
        q1 = scr[6 * nch:7 * nch]
        r0v, r1v, sem = scr[7 * nch], scr[7 * nch + 1], scr[7 * nch + 2]

        w = lax.axis_index("s") * nc + lax.axis_index("c")
        row0 = w * nch
        base = w * cpw

        for j in range(nch):
            pltpu.sync_copy(x_hbm.at[row0 + j], xv[j])

        for j in range(nch):
            for i in range(CW // NLANE):
                sl = pl.ds(i * NLANE, NLANE)
                v = xv[j][sl]
                d = v + v
                i0[j][sl] = d
                i1[j][sl] = d + 1

        cps = []
        for j in range(nch):
            cps.append(pltpu.async_copy(hf_hbm.at[i0[j]], b0[j], sem))
            cps.append(pltpu.async_copy(hf_hbm.at[i1[j]], b1[j], sem))
            cps.append(pltpu.async_copy(pf2_hbm.at[i0[j]], q0[j], sem))
            cps.append(pltpu.async_copy(pf2_hbm.at[i1[j]], q1[j], sem))
        for c in cps:
            c.wait()

        cps = []
        for j in range(nch):
            cps.append(pltpu.async_copy(ep_hbm.at[b0[j]],
                                        r0v.at[pl.ds(j * CW, CW)], sem))
            cps.append(pltpu.async_copy(ep_hbm.at[b1[j]],
                                        r1v.at[pl.ds(j * CW, CW)], sem))
        for c in cps:
            c.wait()

        cps = [pltpu.async_copy(r0v, r0_out.at[pl.ds(base, cpw)], sem),
               pltpu.async_copy(r1v, r1_out.at[pl.ds(base, cpw)], sem)]
        for j in range(nch):
            cps.append(pltpu.async_copy(
                q0[j], q0_out.at[pl.ds(base + j * CW, CW)], sem))
            cps.append(pltpu.async_copy(
                q1[j], q1_out.at[pl.ds(base + j * CW, CW)], sem))
        for c in cps:
            c.wait()

    return body(x2d, hf, pf2, ep)


def _tc_body(q0_ref, q1_ref, r0_ref, r1_ref, w1_ref, w2_ref, o_ref, wct_ref):
    @pl.when(pl.program_id(0) == 0)
    def _():
        wct_ref[0:25, :] = lax.dot_general(
            w1_ref[...], w2_ref[...], (((0,), (1,)), ((), ())),
            preferred_element_type=jnp.float32,
            precision=lax.Precision.HIGHEST)
        wct_ref[25:EPAD, :] = jnp.zeros((EPAD - 25, 300), jnp.float32)
    emb = q0_ref[...] * r0_ref[...] + q1_ref[...] * r1_ref[...]
    logits = jnp.dot(emb, wct_ref[...],
                     preferred_element_type=jnp.float32,
                     precision=lax.Precision.HIGHEST)
    m = jnp.max(logits, axis=1, keepdims=True)
    s = logits - m
    o_ref[...] = s - jnp.log(jnp.sum(jnp.exp(s), axis=1, keepdims=True))


def _tc_mlp(q0, q1, r0, r1, w1, w2):
    br = 4096
    grid = B // br
    return pl.pallas_call(
        _tc_body,
        grid=(grid,),
        in_specs=[
            pl.BlockSpec((br, 1), lambda i: (i, 0)),
            pl.BlockSpec((br, 1), lambda i: (i, 0)),
            pl.BlockSpec((br, EPAD), lambda i: (i, 0)),
            pl.BlockSpec((br, EPAD), lambda i: (i, 0)),
            pl.BlockSpec((128, 25), lambda i: (0, 0)),
            pl.BlockSpec((300, 128), lambda i: (0, 0)),
        ],
        out_specs=pl.BlockSpec((br, 300), lambda i: (i, 0)),
        out_shape=jax.ShapeDtypeStruct((B, 300), jnp.float32),
        scratch_shapes=[pltpu.VMEM((EPAD, 300), jnp.float32)],
    )(q0, q1, r0, r1, w1, w2)


def kernel(x, H, P, E, W1, W2):
    x = x.astype(jnp.int32)
    H = H.astype(jnp.int32)
    x2d = x.reshape(128, 128)
    hf = H.reshape(-1)
    pf2 = P.reshape(-1, 1)
    ep = jnp.pad(E, ((0, 0), (0, EPAD - E.shape[1])))
    r0, r1, q0, q1 = _sc_gather(x2d, hf, pf2, ep)
    return _tc_mlp(q0, q1, r0, r1, W1, W2)


# speedup vs baseline: 3.8492x; 3.8492x over previous
import functools

import jax
import jax.numpy as jnp
from jax import lax
from jax.experimental import pallas as pl
from jax.experimental.pallas import tpu as pltpu
from jax.experimental.pallas import tpu_sc as plsc

B = 16384
ED = 25
EW = 128
CW = 128
NLANE = 16


def _sc_gather(x, h0, h1, p0, p1, e128):
    info = plsc.get_sparse_core_info()
    nw = info.num_cores * info.num_subcores
    cpw = B // nw
    nch = cpw // CW
    nc = info.num_cores

    mesh = plsc.VectorSubcoreMesh(core_axis_name="c", subcore_axis_name="s")

    scratch = (
        [pltpu.VMEM((CW,), jnp.int32) for _ in range(nch)]
        + [pltpu.VMEM((CW,), jnp.int32) for _ in range(2 * nch)]
        + [pltpu.VMEM((CW,), jnp.float32) for _ in range(2 * nch)]
        + [pltpu.VMEM((CW, EW), jnp.float32),
           pltpu.VMEM((CW, EW), jnp.float32),
           pltpu.VMEM((CW, EW), jnp.float32),
           pltpu.VMEM((CW, EW), jnp.float32),
           pltpu.SemaphoreType.DMA,
           pltpu.SemaphoreType.DMA]
    )

    @functools.partial(
        pl.kernel,
        out_type=jax.ShapeDtypeStruct((B, EW), jnp.float32),
        mesh=mesh,
        scratch_types=scratch,
    )
    def body(x_hbm, h0_hbm, h1_hbm, p0_hbm, p1_hbm, e_hbm, emb_out, *scr):
        xv = scr[0:nch]
        b0 = scr[nch:2 * nch]
        b1 = scr[2 * nch:3 * nch]
        q0 = scr[3 * nch:4 * nch]
        q1 = scr[4 * nch:5 * nch]
        r0c, r1c = scr[5 * nch], scr[5 * nch + 1]
        embc = (scr[5 * nch + 2], scr[5 * nch + 3])
        gsem, wsem = scr[5 * nch + 4], scr[5 * nch + 5]

        w = lax.axis_index("s") * nc + lax.axis_index("c")
        base = w * cpw

        for j in range(nch):
            pltpu.sync_copy(x_hbm.at[pl.ds(base + j * CW, CW)], xv[j])

        cps = []
        for j in range(nch):
            cps.append(pltpu.async_copy(h0_hbm.at[xv[j]], b0[j], gsem))
            cps.append(pltpu.async_copy(h1_hbm.at[xv[j]], b1[j], gsem))
            cps.append(pltpu.async_copy(p0_hbm.at[xv[j]], q0[j], gsem))
            cps.append(pltpu.async_copy(p1_hbm.at[xv[j]], q1[j], gsem))
        for c in cps:
            c.wait()

        zero16 = jnp.zeros((NLANE,), jnp.float32)
        for eb in embc:
            def zbody(i, _, eb=eb):
                for l in range(EW // NLANE):
                    eb[i, pl.ds(l * NLANE, NLANE)] = zero16
                return 0
            lax.fori_loop(0, CW, zbody, 0)

        wds = [None, None]
        for j in range(nch):
            eb = embc[j % 2]
            if wds[j % 2] is not None:
                wds[j % 2].wait()
            g0 = pltpu.async_copy(e_hbm.at[b0[j]], r0c, gsem)
            g1 = pltpu.async_copy(e_hbm.at[b1[j]], r1c, gsem)
            g0.wait()
            g1.wait()

            def sbody(g, _, eb=eb, jj=j):
                qv0 = q0[jj][pl.ds(g * NLANE, NLANE)]
                qv1 = q1[jj][pl.ds(g * NLANE, NLANE)]
                for t in range(NLANE):
                    i = g * NLANE + t
                    s0 = qv0[t]
                    s1 = qv1[t]
                    for l in range(2):
                        sl = pl.ds(l * NLANE, NLANE)
                        eb[i, sl] = r0c[i, sl] * s0 + r1c[i, sl] * s1
                return 0
            lax.fori_loop(0, CW // NLANE, sbody, 0)

            wds[j % 2] = pltpu.async_copy(
                eb, emb_out.at[pl.ds(base + j * CW, CW), :], wsem)
        for d in wds:
            d.wait()

    return body(x, h0, h1, p0, p1, e128)


def _tc_body(emb_ref, w1_ref, w2_ref, o_ref, wct_ref):
    @pl.when(pl.program_id(0) == 0)
    def _():
        wct_ref[0:ED, :] = lax.dot_general(
            w1_ref[...], w2_ref[...], (((0,), (1,)), ((), ())),
            preferred_element_type=jnp.float32,
            precision=lax.Precision.HIGHEST)
        wct_ref[ED:EW, :] = jnp.zeros((EW - ED, 300), jnp.float32)
    logits = jnp.dot(emb_ref[...], wct_ref[...],
                     preferred_element_type=jnp.float32,
                     precision=lax.Precision.HIGHEST)
    m = jnp.max(logits, axis=1, keepdims=True)
    s = logits - m
    o_ref[...] = s - jnp.log(jnp.sum(jnp.exp(s), axis=1, keepdims=True))


def _tc_mlp(emb, w1, w2):
    br = 4096
    grid = B // br
    return pl.pallas_call(
        _tc_body,
        grid=(grid,),
        in_specs=[
            pl.BlockSpec((br, EW), lambda i: (i, 0)),
            pl.BlockSpec((128, ED), lambda i: (0, 0)),
            pl.BlockSpec((300, 128), lambda i: (0, 0)),
        ],
        out_specs=pl.BlockSpec((br, 300), lambda i: (i, 0)),
        out_shape=jax.ShapeDtypeStruct((B, 300), jnp.float32),
        scratch_shapes=[pltpu.VMEM((EW, 300), jnp.float32)],
    )(emb, w1, w2)


def kernel(x, H, P, E, W1, W2):
    x = x.astype(jnp.int32)
    H = H.astype(jnp.int32)
    h0 = H[:, 0]
    h1 = H[:, 1]
    p0 = P[:, 0]
    p1 = P[:, 1]
    e128 = jnp.pad(E, ((0, 0), (0, EW - E.shape[1])))
    emb = _sc_gather(x, h0, h1, p0, p1, e128)
    return _tc_mlp(emb, W1, W2)


# speedup vs baseline: 4.1601x; 1.0808x over previous
import functools

import jax
import jax.numpy as jnp
from jax import lax
from jax.experimental import pallas as pl
from jax.experimental.pallas import tpu as pltpu
from jax.experimental.pallas import tpu_sc as plsc

B = 16384
ED = 25
EW = 128
CW = 128
NLANE = 16


def _sc_gather(x, h0, h1, p0, p1, e128):
    info = plsc.get_sparse_core_info()
    nw = info.num_cores * info.num_subcores
    cpw = B // nw
    nch = cpw // CW
    nc = info.num_cores

    mesh = plsc.VectorSubcoreMesh(core_axis_name="c", subcore_axis_name="s")

    scratch = (
        [pltpu.VMEM((CW,), jnp.int32) for _ in range(nch)]
        + [pltpu.VMEM((CW,), jnp.int32) for _ in range(2 * nch)]
        + [pltpu.VMEM((CW,), jnp.float32) for _ in range(2 * nch)]
        + [pltpu.VMEM((CW, EW), jnp.float32),
           pltpu.VMEM((CW, EW), jnp.float32),
           pltpu.VMEM((CW, EW), jnp.float32),
           pltpu.VMEM((CW, EW), jnp.float32),
           pltpu.SemaphoreType.DMA,
           pltpu.SemaphoreType.DMA]
    )

    @functools.partial(
        pl.kernel,
        out_type=jax.ShapeDtypeStruct((B, EW), jnp.float32),
        mesh=mesh,
        scratch_types=scratch,
    )
    def body(x_hbm, h0_hbm, h1_hbm, p0_hbm, p1_hbm, e_hbm, emb_out, *scr):
        xv = scr[0:nch]
        b0 = scr[nch:2 * nch]
        b1 = scr[2 * nch:3 * nch]
        q0 = scr[3 * nch:4 * nch]
        q1 = scr[4 * nch:5 * nch]
        r0c, r1c = scr[5 * nch], scr[5 * nch + 1]
        embc = (scr[5 * nch + 2], scr[5 * nch + 3])
        gsem, wsem = scr[5 * nch + 4], scr[5 * nch + 5]

        w = lax.axis_index("s") * nc + lax.axis_index("c")
        base = w * cpw

        for j in range(nch):
            pltpu.sync_copy(x_hbm.at[pl.ds(base + j * CW, CW)], xv[j])

        cps = []
        for j in range(nch):
            cps.append(pltpu.async_copy(h0_hbm.at[xv[j]], b0[j], gsem))
            cps.append(pltpu.async_copy(h1_hbm.at[xv[j]], b1[j], gsem))
            cps.append(pltpu.async_copy(p0_hbm.at[xv[j]], q0[j], gsem))
            cps.append(pltpu.async_copy(p1_hbm.at[xv[j]], q1[j], gsem))
        for c in cps:
            c.wait()

        zero16 = jnp.zeros((NLANE,), jnp.float32)
        for eb in embc:
            def zbody(i, _, eb=eb):
                for l in range(EW // NLANE):
                    eb[i, pl.ds(l * NLANE, NLANE)] = zero16
                return 0
            lax.fori_loop(0, CW, zbody, 0)

        wds = [None, None]
        for j in range(nch):
            eb = embc[j % 2]
            if wds[j % 2] is not None:
                wds[j % 2].wait()
            g0 = pltpu.async_copy(e_hbm.at[b0[j]], r0c, gsem)
            g1 = pltpu.async_copy(e_hbm.at[b1[j]], r1c, gsem)
            g0.wait()
            g1.wait()

            def sbody(g, _, eb=eb, jj=j):
                qv0 = q0[jj][pl.ds(g * NLANE, NLANE)]
                qv1 = q1[jj][pl.ds(g * NLANE, NLANE)]
                for t in range(NLANE):
                    i = g * NLANE + t
                    s0 = qv0[t]
                    s1 = qv1[t]
                    for l in range(2):
                        sl = pl.ds(l * NLANE, NLANE)
                        eb[i, sl] = r0c[i, sl] * s0 + r1c[i, sl] * s1
                return 0
            lax.fori_loop(0, CW // NLANE, sbody, 0)

            wds[j % 2] = pltpu.async_copy(
                eb, emb_out.at[pl.ds(base + j * CW, CW), :], wsem)
        for d in wds:
            d.wait()

    return body(x, h0, h1, p0, p1, e128)


def _tc_body(emb_ref, w1_ref, w2_ref, o_ref, wct_ref):
    @pl.when(pl.program_id(0) == 0)
    def _():
        wct_ref[0:ED, :] = lax.dot_general(
            w1_ref[...], w2_ref[...], (((0,), (1,)), ((), ())),
            preferred_element_type=jnp.float32,
            precision=lax.Precision.HIGHEST)
        wct_ref[ED:EW, :] = jnp.zeros((EW - ED, 300), jnp.float32)
    emb = emb_ref[...]
    wct = wct_ref[...]
    eh = emb.astype(jnp.bfloat16)
    el = (emb - eh.astype(jnp.float32)).astype(jnp.bfloat16)
    wh = wct.astype(jnp.bfloat16)
    wl = (wct - wh.astype(jnp.float32)).astype(jnp.bfloat16)
    dot = functools.partial(jnp.dot, preferred_element_type=jnp.float32)
    logits = dot(eh, wh) + (dot(el, wh) + dot(eh, wl))
    m = jnp.max(logits, axis=1, keepdims=True)
    s = logits - m
    o_ref[...] = s - jnp.log(jnp.sum(jnp.exp(s), axis=1, keepdims=True))


def _tc_mlp(emb, w1, w2):
    br = 8192
    grid = B // br
    return pl.pallas_call(
        _tc_body,
        grid=(grid,),
        in_specs=[
            pl.BlockSpec((br, EW), lambda i: (i, 0)),
            pl.BlockSpec((128, ED), lambda i: (0, 0)),
            pl.BlockSpec((300, 128), lambda i: (0, 0)),
        ],
        out_specs=pl.BlockSpec((br, 300), lambda i: (i, 0)),
        out_shape=jax.ShapeDtypeStruct((B, 300), jnp.float32),
        scratch_shapes=[pltpu.VMEM((EW, 300), jnp.float32)],
    )(emb, w1, w2)


def kernel(x, H, P, E, W1, W2):
    x = x.astype(jnp.int32)
    h0 = H[:, 0].astype(jnp.int32)
    h1 = H[:, 1].astype(jnp.int32)
    p0 = P[:, 0]
    p1 = P[:, 1]
    e128 = jnp.pad(E, ((0, 0), (0, EW - E.shape[1])))
    emb = _sc_gather(x, h0, h1, p0, p1, e128)
    return _tc_mlp(emb, W1, W2)


# speedup vs baseline: 5.6089x; 1.3483x over previous
import functools

import jax
import jax.numpy as jnp
from jax import lax
from jax.experimental import pallas as pl
from jax.experimental.pallas import tpu as pltpu
from jax.experimental.pallas import tpu_sc as plsc

B = 16384
ED = 25
EW = 128
CW = 128
NLANE = 16


def _sc_gather(x, h0, h1, p0, p1, e128):
    info = plsc.get_sparse_core_info()
    nw = info.num_cores * info.num_subcores
    cpw = B // nw
    nch = cpw // CW
    nc = info.num_cores

    mesh = plsc.VectorSubcoreMesh(core_axis_name="c", subcore_axis_name="s")

    scratch = (
        [pltpu.VMEM((CW,), jnp.int32) for _ in range(nch)]
        + [pltpu.VMEM((CW,), jnp.int32) for _ in range(2 * nch)]
        + [pltpu.VMEM((CW,), jnp.float32) for _ in range(2 * nch)]
        + [pltpu.VMEM((CW, EW), jnp.float32),
           pltpu.VMEM((CW, EW), jnp.float32),
           pltpu.VMEM((CW, EW), jnp.float32),
           pltpu.VMEM((CW, EW), jnp.float32),
           pltpu.SemaphoreType.DMA,
           pltpu.SemaphoreType.DMA]
    )

    @functools.partial(
        pl.kernel,
        out_type=jax.ShapeDtypeStruct((B, EW), jnp.float32),
        mesh=mesh,
        scratch_types=scratch,
    )
    def body(x_hbm, h0_hbm, h1_hbm, p0_hbm, p1_hbm, e_hbm, emb_out, *scr):
        xv = scr[0:nch]
        b0 = scr[nch:2 * nch]
        b1 = scr[2 * nch:3 * nch]
        q0 = scr[3 * nch:4 * nch]
        q1 = scr[4 * nch:5 * nch]
        r0c, r1c = scr[5 * nch], scr[5 * nch + 1]
        embc = (scr[5 * nch + 2], scr[5 * nch + 3])
        gsem, wsem = scr[5 * nch + 4], scr[5 * nch + 5]

        w = lax.axis_index("s") * nc + lax.axis_index("c")
        base = w * cpw

        for j in range(nch):
            pltpu.sync_copy(x_hbm.at[pl.ds(base + j * CW, CW)], xv[j])

        cps = []
        for j in range(nch):
            cps.append(pltpu.async_copy(h0_hbm.at[xv[j]], b0[j], gsem))
            cps.append(pltpu.async_copy(h1_hbm.at[xv[j]], b1[j], gsem))
            cps.append(pltpu.async_copy(p0_hbm.at[xv[j]], q0[j], gsem))
            cps.append(pltpu.async_copy(p1_hbm.at[xv[j]], q1[j], gsem))
        for c in cps:
            c.wait()

        zero16 = jnp.zeros((NLANE,), jnp.float32)
        for eb in embc:
            def zbody(i, _, eb=eb):
                for l in range(EW // NLANE):
                    eb[i, pl.ds(l * NLANE, NLANE)] = zero16
                return 0
            lax.fori_loop(0, CW, zbody, 0)

        wds = [None, None]
        for j in range(nch):
            eb = embc[j % 2]
            if wds[j % 2] is not None:
                wds[j % 2].wait()
            g0 = pltpu.async_copy(e_hbm.at[b0[j]], r0c, gsem)
            g1 = pltpu.async_copy(e_hbm.at[b1[j]], r1c, gsem)
            g0.wait()
            g1.wait()

            def sbody(g, _, eb=eb, jj=j):
                qv0 = q0[jj][pl.ds(g * NLANE, NLANE)]
                qv1 = q1[jj][pl.ds(g * NLANE, NLANE)]
                for t in range(NLANE):
                    i = g * NLANE + t
                    s0 = qv0[t]
                    s1 = qv1[t]
                    for l in range(2):
                        sl = pl.ds(l * NLANE, NLANE)
                        eb[i, sl] = r0c[i, sl] * s0 + r1c[i, sl] * s1
                return 0
            lax.fori_loop(0, CW // NLANE, sbody, 0)

            wds[j % 2] = pltpu.async_copy(
                eb, emb_out.at[pl.ds(base + j * CW, CW), :], wsem)
        for d in wds:
            d.wait()

    return body(x, h0, h1, p0, p1, e128)


def _tc_body(emb_ref, w1_ref, w2_ref, o_ref, wct_ref):
    @pl.when(pl.program_id(0) == 0)
    def _():
        wct_ref[0:ED, :] = lax.dot_general(
            w1_ref[...], w2_ref[...], (((0,), (1,)), ((), ())),
            preferred_element_type=jnp.float32,
            precision=lax.Precision.HIGHEST)
        wct_ref[ED:EW, :] = jnp.zeros((EW - ED, 300), jnp.float32)
    emb = emb_ref[...]
    wct = wct_ref[...]
    eh = emb.astype(jnp.bfloat16)
    el = (emb - eh.astype(jnp.float32)).astype(jnp.bfloat16)
    wh = wct.astype(jnp.bfloat16)
    wl = (wct - wh.astype(jnp.float32)).astype(jnp.bfloat16)
    dot = functools.partial(
        lax.dot_general,
        dimension_numbers=(((0,), (1,)), ((), ())),
        preferred_element_type=jnp.float32)
    logits = dot(wh, eh) + (dot(wl, eh) + dot(wh, el))
    m = jnp.max(logits, axis=0, keepdims=True)
    s = logits - m
    o_ref[...] = s - jnp.log(jnp.sum(jnp.exp(s), axis=0, keepdims=True))


def _tc_mlp(emb, w1, w2):
    br = 8192
    grid = B // br
    return pl.pallas_call(
        _tc_body,
        grid=(grid,),
        in_specs=[
            pl.BlockSpec((br, EW), lambda i: (i, 0)),
            pl.BlockSpec((128, ED), lambda i: (0, 0)),
            pl.BlockSpec((300, 128), lambda i: (0, 0)),
        ],
        out_specs=pl.BlockSpec((300, br), lambda i: (0, i)),
        out_shape=jax.ShapeDtypeStruct((300, B), jnp.float32),
        scratch_shapes=[pltpu.VMEM((EW, 300), jnp.float32)],
    )(emb, w1, w2)


def kernel(x, H, P, E, W1, W2):
    x = x.astype(jnp.int32)
    h0 = H[:, 0].astype(jnp.int32)
    h1 = H[:, 1].astype(jnp.int32)
    p0 = P[:, 0]
    p1 = P[:, 1]
    e128 = jnp.pad(E, ((0, 0), (0, EW - E.shape[1])))
    emb = _sc_gather(x, h0, h1, p0, p1, e128)
    return _tc_mlp(emb, W1, W2).T


# speedup vs baseline: 6.0349x; 1.0759x over previous
import functools

import jax
import jax.numpy as jnp
from jax import lax
from jax.experimental import pallas as pl
from jax.experimental.pallas import tpu as pltpu
from jax.experimental.pallas import tpu_sc as plsc

B = 16384
ED = 25
EW = 128
CW = 128
NLANE = 16


def _sc_gather(x, h0, h1, p0, p1, e128):
    info = plsc.get_sparse_core_info()
    nw = info.num_cores * info.num_subcores
    cpw = B // nw
    nch = cpw // CW
    nc = info.num_cores

    mesh = plsc.VectorSubcoreMesh(core_axis_name="c", subcore_axis_name="s")

    scratch = (
        [pltpu.VMEM((CW,), jnp.int32) for _ in range(nch)]
        + [pltpu.VMEM((CW,), jnp.int32) for _ in range(2 * nch)]
        + [pltpu.VMEM((CW,), jnp.float32) for _ in range(2 * nch)]
        + [pltpu.VMEM((CW, EW), jnp.float32) for _ in range(4)]
        + [pltpu.VMEM((CW, EW), jnp.float32),
           pltpu.VMEM((CW, EW), jnp.float32),
           pltpu.SemaphoreType.DMA,
           pltpu.SemaphoreType.DMA,
           pltpu.SemaphoreType.DMA,
           pltpu.SemaphoreType.DMA]
    )

    @functools.partial(
        pl.kernel,
        out_type=jax.ShapeDtypeStruct((B, EW), jnp.float32),
        mesh=mesh,
        scratch_types=scratch,
    )
    def body(x_hbm, h0_hbm, h1_hbm, p0_hbm, p1_hbm, e_hbm, emb_out, *scr):
        xv = scr[0:nch]
        b0 = scr[nch:2 * nch]
        b1 = scr[2 * nch:3 * nch]
        q0 = scr[3 * nch:4 * nch]
        q1 = scr[4 * nch:5 * nch]
        r0c = scr[5 * nch:5 * nch + 2]
        r1c = scr[5 * nch + 2:5 * nch + 4]
        embc = (scr[5 * nch + 4], scr[5 * nch + 5])
        xsem, gsem, esem, wsem = scr[5 * nch + 6:5 * nch + 10]

        w = lax.axis_index("s") * nc + lax.axis_index("c")
        base = w * cpw

        xds = [pltpu.async_copy(x_hbm.at[pl.ds(base + j * CW, CW)],
                                xv[j], xsem) for j in range(nch)]
        gds = []
        for j in range(nch):
            xds[j].wait()
            gds.append([pltpu.async_copy(h0_hbm.at[xv[j]], b0[j], gsem),
                        pltpu.async_copy(h1_hbm.at[xv[j]], b1[j], gsem),
                        pltpu.async_copy(p0_hbm.at[xv[j]], q0[j], gsem),
                        pltpu.async_copy(p1_hbm.at[xv[j]], q1[j], gsem)])

        zero16 = jnp.zeros((NLANE,), jnp.float32)
        for eb in embc:
            def zbody(i, _, eb=eb):
                for l in range(EW // NLANE):
                    eb[i, pl.ds(l * NLANE, NLANE)] = zero16
                return 0
            lax.fori_loop(0, CW, zbody, 0)

        def fire_e(j):
            for c in gds[j][:2]:
                c.wait()
            return (pltpu.async_copy(e_hbm.at[b0[j]], r0c[j % 2], esem),
                    pltpu.async_copy(e_hbm.at[b1[j]], r1c[j % 2], esem))

        eds = {0: fire_e(0)}
        wds = [None, None]
        for j in range(nch):
            if j + 1 < nch:
                eds[j + 1] = fire_e(j + 1)
            for c in eds[j]:
                c.wait()
            for c in gds[j][2:]:
                c.wait()
            eb = embc[j % 2]
            if wds[j % 2] is not None:
                wds[j % 2].wait()
            ra, rb = r0c[j % 2], r1c[j % 2]

            def sbody(g, _, eb=eb, ra=ra, rb=rb, jj=j):
                qv0 = q0[jj][pl.ds(g * NLANE, NLANE)]
                qv1 = q1[jj][pl.ds(g * NLANE, NLANE)]
                for t in range(NLANE):
                    i = g * NLANE + t
                    s0 = qv0[t]
                    s1 = qv1[t]
                    for l in range(2):
                        sl = pl.ds(l * NLANE, NLANE)
                        eb[i, sl] = ra[i, sl] * s0 + rb[i, sl] * s1
                return 0
            lax.fori_loop(0, CW // NLANE, sbody, 0)

            wds[j % 2] = pltpu.async_copy(
                eb, emb_out.at[pl.ds(base + j * CW, CW), :], wsem)
        for d in wds:
            d.wait()

    return body(x, h0, h1, p0, p1, e128)


def _tc_body(emb_ref, w1_ref, w2_ref, o_ref, wct_ref):
    @pl.when(pl.program_id(0) == 0)
    def _():
        wct_ref[0:ED, :] = lax.dot_general(
            w1_ref[...], w2_ref[...], (((0,), (1,)), ((), ())),
            preferred_element_type=jnp.float32,
            precision=lax.Precision.HIGHEST)
        wct_ref[ED:EW, :] = jnp.zeros((EW - ED, 300), jnp.float32)
    emb = emb_ref[...]
    wct = wct_ref[...]
    eh = emb.astype(jnp.bfloat16)
    el = (emb - eh.astype(jnp.float32)).astype(jnp.bfloat16)
    wh = wct.astype(jnp.bfloat16)
    wl = (wct - wh.astype(jnp.float32)).astype(jnp.bfloat16)
    dot = functools.partial(
        lax.dot_general,
        dimension_numbers=(((0,), (1,)), ((), ())),
        preferred_element_type=jnp.float32)
    logits = dot(wh, eh) + (dot(wl, eh) + dot(wh, el))
    m = jnp.max(logits, axis=0, keepdims=True)
    s = logits - m
    o_ref[...] = s - jnp.log(jnp.sum(jnp.exp(s), axis=0, keepdims=True))


def _tc_mlp(emb, w1, w2):
    br = 8192
    grid = B // br
    return pl.pallas_call(
        _tc_body,
        grid=(grid,),
        in_specs=[
            pl.BlockSpec((br, EW), lambda i: (i, 0)),
            pl.BlockSpec((128, ED), lambda i: (0, 0)),
            pl.BlockSpec((300, 128), lambda i: (0, 0)),
        ],
        out_specs=pl.BlockSpec((300, br), lambda i: (0, i)),
        out_shape=jax.ShapeDtypeStruct((300, B), jnp.float32),
        scratch_shapes=[pltpu.VMEM((EW, 300), jnp.float32)],
    )(emb, w1, w2)


def kernel(x, H, P, E, W1, W2):
    x = x.astype(jnp.int32)
    h0 = H[:, 0].astype(jnp.int32)
    h1 = H[:, 1].astype(jnp.int32)
    p0 = P[:, 0]
    p1 = P[:, 1]
    e128 = jnp.pad(E, ((0, 0), (0, EW - E.shape[1])))
    emb = _sc_gather(x, h0, h1, p0, p1, e128)
    return _tc_mlp(emb, W1, W2).T


# speedup vs baseline: 6.2339x; 1.0330x over previous
import functools

import jax
import jax.numpy as jnp
from jax import lax
from jax.experimental import pallas as pl
from jax.experimental.pallas import tpu as pltpu
from jax.experimental.pallas import tpu_sc as plsc

B = 16384
ED = 25
EW = 128
EC = 32
CW = 128
NLANE = 16


def _sc_gather(x, h0, h1, p0, p1, e128):
    info = plsc.get_sparse_core_info()
    nw = info.num_cores * info.num_subcores
    cpw = B // nw
    nch = cpw // CW
    nc = info.num_cores

    mesh = plsc.VectorSubcoreMesh(core_axis_name="c", subcore_axis_name="s")

    scratch = (
        [pltpu.VMEM((CW,), jnp.int32) for _ in range(nch)]
        + [pltpu.VMEM((CW,), jnp.int32) for _ in range(2 * nch)]
        + [pltpu.VMEM((CW,), jnp.float32) for _ in range(2 * nch)]
        + [pltpu.VMEM((CW, EW), jnp.float32) for _ in range(4)]
        + [pltpu.VMEM((CW, EC), jnp.float32),
           pltpu.VMEM((CW, EC), jnp.float32),
           pltpu.SemaphoreType.DMA,
           pltpu.SemaphoreType.DMA,
           pltpu.SemaphoreType.DMA,
           pltpu.SemaphoreType.DMA]
    )

    @functools.partial(
        pl.kernel,
        out_type=jax.ShapeDtypeStruct((B, EC), jnp.float32),
        mesh=mesh,
        scratch_types=scratch,
    )
    def body(x_hbm, h0_hbm, h1_hbm, p0_hbm, p1_hbm, e_hbm, emb_out, *scr):
        xv = scr[0:nch]
        b0 = scr[nch:2 * nch]
        b1 = scr[2 * nch:3 * nch]
        q0 = scr[3 * nch:4 * nch]
        q1 = scr[4 * nch:5 * nch]
        r0c = scr[5 * nch:5 * nch + 2]
        r1c = scr[5 * nch + 2:5 * nch + 4]
        embc = (scr[5 * nch + 4], scr[5 * nch + 5])
        xsem, gsem, esem, wsem = scr[5 * nch + 6:5 * nch + 10]

        w = lax.axis_index("s") * nc + lax.axis_index("c")
        base = w * cpw

        xds = [pltpu.async_copy(x_hbm.at[pl.ds(base + j * CW, CW)],
                                xv[j], xsem) for j in range(nch)]
        gds = []
        for j in range(nch):
            xds[j].wait()
            gds.append([pltpu.async_copy(h0_hbm.at[xv[j]], b0[j], gsem),
                        pltpu.async_copy(h1_hbm.at[xv[j]], b1[j], gsem),
                        pltpu.async_copy(p0_hbm.at[xv[j]], q0[j], gsem),
                        pltpu.async_copy(p1_hbm.at[xv[j]], q1[j], gsem)])

        def fire_e(j):
            for c in gds[j][:2]:
                c.wait()
            return (pltpu.async_copy(e_hbm.at[b0[j]], r0c[j % 2], esem),
                    pltpu.async_copy(e_hbm.at[b1[j]], r1c[j % 2], esem))

        eds = {0: fire_e(0)}
        wds = [None, None]
        for j in range(nch):
            if j + 1 < nch:
                eds[j + 1] = fire_e(j + 1)
            for c in eds[j]:
                c.wait()
            for c in gds[j][2:]:
                c.wait()
            eb = embc[j % 2]
            if wds[j % 2] is not None:
                wds[j % 2].wait()
            ra, rb = r0c[j % 2], r1c[j % 2]

            def sbody(g, _, eb=eb, ra=ra, rb=rb, jj=j):
                qv0 = q0[jj][pl.ds(g * NLANE, NLANE)]
                qv1 = q1[jj][pl.ds(g * NLANE, NLANE)]
                for t in range(NLANE):
                    i = g * NLANE + t
                    s0 = qv0[t]
                    s1 = qv1[t]
                    for l in range(2):
                        sl = pl.ds(l * NLANE, NLANE)
                        eb[i, sl] = ra[i, sl] * s0 + rb[i, sl] * s1
                return 0
            lax.fori_loop(0, CW // NLANE, sbody, 0)

            wds[j % 2] = pltpu.async_copy(
                eb, emb_out.at[pl.ds(base + j * CW, CW), :], wsem)
        for d in wds:
            d.wait()

    return body(x, h0, h1, p0, p1, e128)


def _tc_body(emb_ref, w1_ref, w2_ref, o_ref, wct_ref):
    @pl.when(pl.program_id(0) == 0)
    def _():
        wct_ref[0:ED, :] = lax.dot_general(
            w1_ref[...], w2_ref[...], (((0,), (1,)), ((), ())),
            preferred_element_type=jnp.float32,
            precision=lax.Precision.HIGHEST)
        wct_ref[ED:EC, :] = jnp.zeros((EC - ED, 300), jnp.float32)
    emb = emb_ref[...]
    wct = wct_ref[...]
    eh = emb.astype(jnp.bfloat16)
    el = (emb - eh.astype(jnp.float32)).astype(jnp.bfloat16)
    wh = wct.astype(jnp.bfloat16)
    wl = (wct - wh.astype(jnp.float32)).astype(jnp.bfloat16)
    dot = functools.partial(
        lax.dot_general,
        dimension_numbers=(((0,), (1,)), ((), ())),
        preferred_element_type=jnp.float32)
    logits = dot(wh, eh) + (dot(wl, eh) + dot(wh, el))
    m = jnp.max(logits, axis=0, keepdims=True)
    s = logits - m
    o_ref[...] = s - jnp.log(jnp.sum(jnp.exp(s), axis=0, keepdims=True))


def _tc_mlp(emb, w1, w2):
    br = 4096
    grid = B // br
    return pl.pallas_call(
        _tc_body,
        grid=(grid,),
        in_specs=[
            pl.BlockSpec((br, EC), lambda i: (i, 0)),
            pl.BlockSpec((128, ED), lambda i: (0, 0)),
            pl.BlockSpec((300, 128), lambda i: (0, 0)),
        ],
        out_specs=pl.BlockSpec((300, br), lambda i: (0, i)),
        out_shape=jax.ShapeDtypeStruct((300, B), jnp.float32),
        scratch_shapes=[pltpu.VMEM((EC, 300), jnp.float32)],
    )(emb, w1, w2)


def kernel(x, H, P, E, W1, W2):
    x = x.astype(jnp.int32)
    h0 = H[:, 0].astype(jnp.int32)
    h1 = H[:, 1].astype(jnp.int32)
    p0 = P[:, 0]
    p1 = P[:, 1]
    e128 = jnp.pad(E, ((0, 0), (0, EW - E.shape[1])))
    emb = _sc_gather(x, h0, h1, p0, p1, e128)
    return _tc_mlp(emb, W1, W2).T
